# baseline (speedup 1.0000x reference)
import functools

import jax
import jax.numpy as jnp
from jax.experimental import pallas as pl
from jax.experimental.pallas import tpu as pltpu

_B, _S, _D = 4, 2048, 1024
_T = _S // 2
_ROWS = 1024
_NT = _S // _ROWS


def _flow_kernel(sent_lo_ref, sent_hi_ref, ent_lo_ref, ent_hi_ref,
                 w_ref, bias_ref,
                 hat_out, a_out, loss_out,
                 loss_acc, last_d2, cnt):
    b = pl.program_id(0)
    t = pl.program_id(1)

    @pl.when(jnp.logical_and(b == 0, t == 0))
    def _init_loss():
        loss_out[...] = jnp.zeros((1, 128), jnp.float32)

    @pl.when(t == 0)
    def _init_batch():
        loss_acc[0] = 0.0
        last_d2[0] = 0.0
        cnt[0] = 0

    x = jnp.concatenate([sent_lo_ref[0], sent_hi_ref[0]], axis=1)
    e = jnp.concatenate([ent_lo_ref[0], ent_hi_ref[0]], axis=1)

    mm = jax.lax.dot_general(
        e.astype(jnp.bfloat16), w_ref[...],
        dimension_numbers=(((1,), (1,)), ((), ())),
        preferred_element_type=jnp.float32)
    mm_s = pltpu.roll(mm, _ROWS - 1, 0)
    x_s = pltpu.roll(x, _ROWS - 1, 0)

    hat = x + mm_s + bias_ref[...]
    diff = hat - x_s

    rowsum = jnp.sum(x, axis=1, keepdims=True)
    ids = jax.lax.broadcasted_iota(jnp.int32, (_ROWS, 1), 0)
    maskv = (rowsum != 0.0) & (ids % 2 == 0)
    d2row = jnp.sum(diff * diff, axis=1, keepdims=True)
    loss_acc[0] += jnp.sum(jnp.where(maskv, d2row, 0.0))

    tile_cnt = jnp.sum(maskv.astype(jnp.int32))
    cnt[0] += tile_cnt

    tl = jnp.max(jnp.where(maskv, ids, -1))

    @pl.when(tile_cnt > 0)
    def _track_last():
        sel = ((ids == tl) & maskv).astype(jnp.float32)
        hat_out[0, 0, :] = jnp.sum(hat * sel, axis=0)
        a_out[0, 0, :] = jnp.sum(x_s * sel, axis=0)
        last_d2[0] = jnp.sum(d2row * sel)

    @pl.when(t == _NT - 1)
    def _finish_batch():
        @pl.when(cnt[0] == 0)
        def _fallback():
            hat_out[0, 0, :] = hat[_ROWS - 2, :]
            a_out[0, 0, :] = x_s[_ROWS - 2, :]
            last_d2[0] = 0.0

        loss_out[...] = loss_out[...] + (loss_acc[0] - last_d2[0])


@functools.partial(jax.jit, static_argnames=())
def kernel(sent_emb, entity_emb, W, b):
    bias = b.reshape(1, _D)
    wt_bf16 = W.astype(jnp.bfloat16)

    hat_n, a_n, loss = pl.pallas_call(
        _flow_kernel,
        grid=(_B, _NT),
        in_specs=[
            pl.BlockSpec((1, _ROWS, _D // 2), lambda b_, t_: (b_, t_, 0)),
            pl.BlockSpec((1, _ROWS, _D // 2), lambda b_, t_: (b_, t_, 1)),
            pl.BlockSpec((1, _ROWS, _D // 2), lambda b_, t_: (b_, t_, 0)),
            pl.BlockSpec((1, _ROWS, _D // 2), lambda b_, t_: (b_, t_, 1)),
            pl.BlockSpec((_D, _D), lambda b_, t_: (0, 0)),
            pl.BlockSpec((1, _D), lambda b_, t_: (0, 0)),
        ],
        out_specs=[
            pl.BlockSpec((1, 1, _D), lambda b_, t_: (b_, 0, 0)),
            pl.BlockSpec((1, 1, _D), lambda b_, t_: (b_, 0, 0)),
            pl.BlockSpec((1, 128), lambda b_, t_: (0, 0)),
        ],
        out_shape=[
            jax.ShapeDtypeStruct((_B, 1, _D), jnp.float32),
            jax.ShapeDtypeStruct((_B, 1, _D), jnp.float32),
            jax.ShapeDtypeStruct((1, 128), jnp.float32),
        ],
        scratch_shapes=[
            pltpu.SMEM((1,), jnp.float32),
            pltpu.SMEM((1,), jnp.float32),
            pltpu.SMEM((1,), jnp.int32),
        ],
    )(sent_emb, sent_emb, entity_emb, entity_emb, wt_bf16, bias)

    return (hat_n[:, 0, :], a_n[:, 0, :], loss[0, 0])


# speedup vs baseline: 1.0034x; 1.0034x over previous
import functools

import jax
import jax.numpy as jnp
from jax.experimental import pallas as pl
from jax.experimental.pallas import tpu as pltpu

_B, _S, _D = 4, 2048, 1024
_T = _S // 2
_ROWS = 1024
_NT = _S // _ROWS


def _flow_kernel(sent_ref, ent_ref, w_ref, bias_ref,
                 hat_out, a_out, loss_out,
                 loss_acc, last_d2, cnt):
    b = pl.program_id(0)
    t = pl.program_id(1)

    @pl.when(jnp.logical_and(b == 0, t == 0))
    def _init_loss():
        loss_out[...] = jnp.zeros((1, 128), jnp.float32)

    @pl.when(t == 0)
    def _init_batch():
        loss_acc[0] = 0.0
        last_d2[0] = 0.0
        cnt[0] = 0

    x = sent_ref[0]
    e = ent_ref[0]

    mm = jax.lax.dot_general(
        e.astype(jnp.bfloat16), w_ref[...],
        dimension_numbers=(((1,), (1,)), ((), ())),
        preferred_element_type=jnp.float32)
    mm_s = pltpu.roll(mm, _ROWS - 1, 0)
    x_s = pltpu.roll(x, _ROWS - 1, 0)

    hat = x + mm_s + bias_ref[...]
    diff = hat - x_s

    rowsum = jnp.sum(x, axis=1, keepdims=True)
    ids = jax.lax.broadcasted_iota(jnp.int32, (_ROWS, 1), 0)
    maskv = (rowsum != 0.0) & (ids % 2 == 0)
    d2row = jnp.sum(diff * diff, axis=1, keepdims=True)
    loss_acc[0] += jnp.sum(jnp.where(maskv, d2row, 0.0))

    tile_cnt = jnp.sum(maskv.astype(jnp.int32))
    cnt[0] += tile_cnt

    tl = jnp.max(jnp.where(maskv, ids, -1))

    @pl.when(tile_cnt > 0)
    def _track_last():
        sel = ((ids == tl) & maskv).astype(jnp.float32)
        hat_out[0, 0, :] = jnp.sum(hat * sel, axis=0)
        a_out[0, 0, :] = jnp.sum(x_s * sel, axis=0)
        last_d2[0] = jnp.sum(d2row * sel)

    @pl.when(t == _NT - 1)
    def _finish_batch():
        @pl.when(cnt[0] == 0)
        def _fallback():
            hat_out[0, 0, :] = hat[_ROWS - 2, :]
            a_out[0, 0, :] = x_s[_ROWS - 2, :]
            last_d2[0] = 0.0

        loss_out[...] = loss_out[...] + (loss_acc[0] - last_d2[0])


@functools.partial(jax.jit, static_argnames=())
def kernel(sent_emb, entity_emb, W, b):
    bias = b.reshape(1, _D)
    wt_bf16 = W.astype(jnp.bfloat16)

    hat_n, a_n, loss = pl.pallas_call(
        _flow_kernel,
        grid=(_B, _NT),
        in_specs=[
            pl.BlockSpec((1, _ROWS, _D), lambda b_, t_: (b_, t_, 0)),
            pl.BlockSpec((1, _ROWS, _D), lambda b_, t_: (b_, t_, 0)),
            pl.BlockSpec((_D, _D), lambda b_, t_: (0, 0)),
            pl.BlockSpec((1, _D), lambda b_, t_: (0, 0)),
        ],
        out_specs=[
            pl.BlockSpec((1, 1, _D), lambda b_, t_: (b_, 0, 0)),
            pl.BlockSpec((1, 1, _D), lambda b_, t_: (b_, 0, 0)),
            pl.BlockSpec((1, 128), lambda b_, t_: (0, 0)),
        ],
        out_shape=[
            jax.ShapeDtypeStruct((_B, 1, _D), jnp.float32),
            jax.ShapeDtypeStruct((_B, 1, _D), jnp.float32),
            jax.ShapeDtypeStruct((1, 128), jnp.float32),
        ],
        scratch_shapes=[
            pltpu.SMEM((1,), jnp.float32),
            pltpu.SMEM((1,), jnp.float32),
            pltpu.SMEM((1,), jnp.int32),
        ],
    )(sent_emb, entity_emb, wt_bf16, bias)

    return (hat_n[:, 0, :], a_n[:, 0, :], loss[0, 0])


# speedup vs baseline: 1.1759x; 1.1719x over previous
import functools

import jax
import jax.numpy as jnp
from jax.experimental import pallas as pl
from jax.experimental.pallas import tpu as pltpu

_B, _S, _D = 4, 2048, 1024
_T = _S // 2
_ROWS = 1024
_NT = _S // _ROWS


def _flow_kernel(sent_ref, ent_ref, w_ref, bias_ref,
                 hat_out, a_out, loss_out,
                 loss_acc, last_d2, cnt):
    b = pl.program_id(0)
    t = pl.program_id(1)

    @pl.when(jnp.logical_and(b == 0, t == 0))
    def _init_loss():
        loss_out[...] = jnp.zeros((1, 128), jnp.float32)

    @pl.when(t == 0)
    def _init_batch():
        loss_acc[0] = 0.0
        last_d2[0] = 0.0
        cnt[0] = 0

    x = sent_ref[0]
    e = ent_ref[0]

    mm = jax.lax.dot_general(
        e.astype(jnp.bfloat16), w_ref[...],
        dimension_numbers=(((1,), (1,)), ((), ())),
        preferred_element_type=jnp.float32)
    g_s = pltpu.roll(mm - x, _ROWS - 1, 0)
    diff = x + g_s + bias_ref[...]

    rowsum = jnp.sum(x, axis=1, keepdims=True)
    ids = jax.lax.broadcasted_iota(jnp.int32, (_ROWS, 1), 0)
    maskv = (rowsum != 0.0) & (ids % 2 == 0)
    d2row = jnp.sum(diff * diff, axis=1, keepdims=True)
    loss_acc[0] += jnp.sum(jnp.where(maskv, d2row, 0.0))

    tile_cnt = jnp.sum(maskv.astype(jnp.int32))
    cnt[0] += tile_cnt

    tl = jnp.max(jnp.where(maskv, ids, -1))

    def _emit_rows(q_row, a_row, e_row, store_d2):
        mm_row = jax.lax.dot_general(
            e_row.astype(jnp.bfloat16), w_ref[...],
            dimension_numbers=(((1,), (1,)), ((), ())),
            preferred_element_type=jnp.float32)
        hat_row = q_row + mm_row + bias_ref[...]
        hat_out[0, 0, :] = hat_row[0]
        a_out[0, 0, :] = a_row[0]
        dd = hat_row - a_row
        last_d2[0] = jnp.sum(dd * dd) if store_d2 else 0.0

    @pl.when(tile_cnt > 0)
    def _track_last():
        _emit_rows(sent_ref[0, pl.ds(tl, 1), :],
                   sent_ref[0, pl.ds(tl + 1, 1), :],
                   ent_ref[0, pl.ds(tl + 1, 1), :],
                   store_d2=True)

    @pl.when(t == _NT - 1)
    def _finish_batch():
        @pl.when(cnt[0] == 0)
        def _fallback():
            _emit_rows(sent_ref[0, _ROWS - 2:_ROWS - 1, :],
                       sent_ref[0, _ROWS - 1:_ROWS, :],
                       ent_ref[0, _ROWS - 1:_ROWS, :],
                       store_d2=False)

        loss_out[...] = loss_out[...] + (loss_acc[0] - last_d2[0])


@functools.partial(jax.jit, static_argnames=())
def kernel(sent_emb, entity_emb, W, b):
    bias = b.reshape(1, _D)
    wt_bf16 = W.astype(jnp.bfloat16)

    hat_n, a_n, loss = pl.pallas_call(
        _flow_kernel,
        grid=(_B, _NT),
        in_specs=[
            pl.BlockSpec((1, _ROWS, _D), lambda b_, t_: (b_, t_, 0)),
            pl.BlockSpec((1, _ROWS, _D), lambda b_, t_: (b_, t_, 0)),
            pl.BlockSpec((_D, _D), lambda b_, t_: (0, 0)),
            pl.BlockSpec((1, _D), lambda b_, t_: (0, 0)),
        ],
        out_specs=[
            pl.BlockSpec((1, 1, _D), lambda b_, t_: (b_, 0, 0)),
            pl.BlockSpec((1, 1, _D), lambda b_, t_: (b_, 0, 0)),
            pl.BlockSpec((1, 128), lambda b_, t_: (0, 0)),
        ],
        out_shape=[
            jax.ShapeDtypeStruct((_B, 1, _D), jnp.float32),
            jax.ShapeDtypeStruct((_B, 1, _D), jnp.float32),
            jax.ShapeDtypeStruct((1, 128), jnp.float32),
        ],
        scratch_shapes=[
            pltpu.SMEM((1,), jnp.float32),
            pltpu.SMEM((1,), jnp.float32),
            pltpu.SMEM((1,), jnp.int32),
        ],
    )(sent_emb, entity_emb, wt_bf16, bias)

    return (hat_n[:, 0, :], a_n[:, 0, :], loss[0, 0])


# speedup vs baseline: 1.2122x; 1.0309x over previous
import functools

import jax
import jax.numpy as jnp
from jax.experimental import pallas as pl
from jax.experimental.pallas import tpu as pltpu

_B, _S, _D = 4, 2048, 1024
_T = _S // 2
_ROWS = 1024
_NT = _S // _ROWS


def _flow_kernel(sent_ref, ent_ref, w_ref, bias_ref,
                 hat_out, a_out, loss_out,
                 loss_acc, last_d2, cnt):
    b = pl.program_id(0)
    t = pl.program_id(1)

    @pl.when(jnp.logical_and(b == 0, t == 0))
    def _init_loss():
        loss_out[...] = jnp.zeros((1, 128), jnp.float32)

    @pl.when(t == 0)
    def _init_batch():
        loss_acc[0] = 0.0
        last_d2[0] = 0.0
        cnt[0] = 0

    x = sent_ref[0]
    e = ent_ref[0]

    mm = jax.lax.dot_general(
        e.astype(jnp.bfloat16), w_ref[...],
        dimension_numbers=(((1,), (0,)), ((), ())),
        preferred_element_type=jnp.float32)
    g_s = pltpu.roll(mm - x, _ROWS - 1, 0)
    diff = x + g_s + bias_ref[...]

    rowsum = jnp.sum(x, axis=1, keepdims=True)
    ids = jax.lax.broadcasted_iota(jnp.int32, (_ROWS, 1), 0)
    maskv = (rowsum != 0.0) & (ids % 2 == 0)
    d2row = jnp.sum(diff * diff, axis=1, keepdims=True)
    loss_acc[0] += jnp.sum(jnp.where(maskv, d2row, 0.0))

    tile_cnt = jnp.sum(maskv.astype(jnp.int32))
    cnt[0] += tile_cnt

    tl = jnp.max(jnp.where(maskv, ids, -1))

    def _emit_rows(q_row, a_row, e_row, store_d2):
        mm_row = jax.lax.dot_general(
            e_row.astype(jnp.bfloat16), w_ref[...],
            dimension_numbers=(((1,), (0,)), ((), ())),
            preferred_element_type=jnp.float32)
        hat_row = q_row + mm_row + bias_ref[...]
        hat_out[0, 0, :] = hat_row[0]
        a_out[0, 0, :] = a_row[0]
        dd = hat_row - a_row
        last_d2[0] = jnp.sum(dd * dd) if store_d2 else 0.0

    @pl.when(tile_cnt > 0)
    def _track_last():
        _emit_rows(sent_ref[0, pl.ds(tl, 1), :],
                   sent_ref[0, pl.ds(tl + 1, 1), :],
                   ent_ref[0, pl.ds(tl + 1, 1), :],
                   store_d2=True)

    @pl.when(t == _NT - 1)
    def _finish_batch():
        @pl.when(cnt[0] == 0)
        def _fallback():
            _emit_rows(sent_ref[0, _ROWS - 2:_ROWS - 1, :],
                       sent_ref[0, _ROWS - 1:_ROWS, :],
                       ent_ref[0, _ROWS - 1:_ROWS, :],
                       store_d2=False)

        loss_out[...] = loss_out[...] + (loss_acc[0] - last_d2[0])


@functools.partial(jax.jit, static_argnames=())
def kernel(sent_emb, entity_emb, W, b):
    bias = b.reshape(1, _D)
    wt_bf16 = W.T.astype(jnp.bfloat16)

    hat_n, a_n, loss = pl.pallas_call(
        _flow_kernel,
        grid=(_B, _NT),
        in_specs=[
            pl.BlockSpec((1, _ROWS, _D), lambda b_, t_: (b_, t_, 0)),
            pl.BlockSpec((1, _ROWS, _D), lambda b_, t_: (b_, t_, 0)),
            pl.BlockSpec((_D, _D), lambda b_, t_: (0, 0)),
            pl.BlockSpec((1, _D), lambda b_, t_: (0, 0)),
        ],
        out_specs=[
            pl.BlockSpec((1, 1, _D), lambda b_, t_: (b_, 0, 0)),
            pl.BlockSpec((1, 1, _D), lambda b_, t_: (b_, 0, 0)),
            pl.BlockSpec((1, 128), lambda b_, t_: (0, 0)),
        ],
        out_shape=[
            jax.ShapeDtypeStruct((_B, 1, _D), jnp.float32),
            jax.ShapeDtypeStruct((_B, 1, _D), jnp.float32),
            jax.ShapeDtypeStruct((1, 128), jnp.float32),
        ],
        scratch_shapes=[
            pltpu.SMEM((1,), jnp.float32),
            pltpu.SMEM((1,), jnp.float32),
            pltpu.SMEM((1,), jnp.int32),
        ],
    )(sent_emb, entity_emb, wt_bf16, bias)

    return (hat_n[:, 0, :], a_n[:, 0, :], loss[0, 0])
